# baseline (speedup 1.0000x reference)
import functools

import jax
import jax.numpy as jnp
from jax import lax
from jax.experimental import pallas as pl
from jax.experimental.pallas import tpu as pltpu
from jax.experimental.pallas import tpu_sc as plsc

DIM = 128
MAX_LEN = 2048
N_POS = 819200

_NC = 2
_NS = 16
_NW = _NC * _NS

_B_PER_W = N_POS // _NW
_G = 128
_NG = _B_PER_W // _G

_NBUF = 5
_LEAD = 2


def _make_sc_gather():
    mesh = plsc.VectorSubcoreMesh(core_axis_name="c", subcore_axis_name="s")

    @functools.partial(
        pl.kernel,
        mesh=mesh,
        out_type=jax.ShapeDtypeStruct((N_POS, DIM), jnp.float32),
        scratch_types=[
            pltpu.VMEM((_NG, _G), jnp.int32),
            pltpu.VMEM((_NBUF, _G, DIM), jnp.float32),
            pltpu.VMEM_SHARED((MAX_LEN, DIM), jnp.float32),
            [pltpu.SemaphoreType.DMA] * _NBUF,
            [pltpu.SemaphoreType.DMA] * _NBUF,
        ],
    )
    def body(table_hbm, pos_hbm, out_hbm, idx_v, rows_v, table_sp, gsems, ssems):
        wid = lax.axis_index("s") * _NC + lax.axis_index("c")
        base = wid * _B_PER_W

        @pl.when(lax.axis_index("s") == 0)
        def _():
            pltpu.sync_copy(table_hbm, table_sp)

        pltpu.sync_copy(pos_hbm.at[pl.ds(wid * _NG, _NG)], idx_v)
        plsc.subcore_barrier()

        def g_copy(j, b):
            return pltpu.make_async_copy(
                table_sp.at[idx_v.at[j]], rows_v.at[b], gsems[b]
            )

        def s_copy(j, b):
            return pltpu.make_async_copy(
                rows_v.at[b], out_hbm.at[pl.ds(base + j * _G, _G)], ssems[b]
            )

        for j in range(_NBUF):
            g_copy(j, j).start()
            if j >= _LEAD:
                c = j - _LEAD
                g_copy(c, c).wait()
                s_copy(c, c).start()

        def step(i, carry):
            j0 = _NBUF + i * _NBUF
            for k in range(_NBUF):
                j = j0 + k
                bl = (k - _LEAD) % _NBUF
                s_copy(j - _NBUF, k).wait()
                g_copy(j, k).start()
                g_copy(j - _LEAD, bl).wait()
                s_copy(j - _LEAD, bl).start()
            return carry

        lax.fori_loop(0, (_NG - _NBUF) // _NBUF, step, 0)

        for t in range(_LEAD):
            j = _NG + t
            bl = (j - _LEAD) % _NBUF
            s_copy(j - _NBUF, j % _NBUF).wait()
            g_copy(j - _LEAD, bl).wait()
            s_copy(j - _LEAD, bl).start()

        for c in range(_NG + _LEAD - _NBUF, _NG):
            s_copy(c, c % _NBUF).wait()

    return body


_sc_gather = _make_sc_gather()


def kernel(pe, pos):
    table = pe.reshape(MAX_LEN, DIM)
    pos2 = pos.reshape(N_POS // _G, _G)
    return _sc_gather(table, pos2)
